# baseline (speedup 1.0000x reference)
import functools

import jax
import jax.numpy as jnp
from jax import lax
from jax.experimental import pallas as pl
from jax.experimental.pallas import tpu as pltpu
from jax.experimental.pallas import tpu_sc as plsc

E = 8
K = 2
D = 1024
F = 2048
N = 2048
A = N * K
T = 128
NT_MAX = A // T + E
P = NT_MAX * T
L = 16
NW = 32


def _mesh():
    return plsc.VectorSubcoreMesh(core_axis_name="c", subcore_axis_name="s")


_SC_PARAMS = pltpu.CompilerParams(needs_layout_passes=False)


def _routing(ei):
    CHS = A // L

    @functools.partial(
        pl.kernel,
        out_type=(jax.ShapeDtypeStruct((L,), jnp.int32),
                  jax.ShapeDtypeStruct((2 * L,), jnp.int32),
                  jax.ShapeDtypeStruct((A,), jnp.int32)),
        mesh=_mesh(),
        compiler_params=_SC_PARAMS,
        scratch_types=[pltpu.VMEM((A,), jnp.int32),
                       pltpu.VMEM((A,), jnp.int32),
                       pltpu.VMEM((L,), jnp.int32),
                       pltpu.VMEM((2 * L,), jnp.int32),
                       pltpu.VMEM((L * E,), jnp.int32),
                       pltpu.VMEM((L * E,), jnp.int32),
                       pltpu.VMEM((L,), jnp.int32),
                       pltpu.SemaphoreType.DMA],
    )
    def k(ei_hbm, meta_hbm, eid_hbm, dst_hbm,
          ei_v, dst_v, meta_v, eid_v, cnt_v, pre_v, tmp_v, sem):
        wid = lax.axis_index("s") * 2 + lax.axis_index("c")
        iota = lax.iota(jnp.int32, L)
        zero_v = jnp.zeros((L,), jnp.int32)

        pltpu.sync_copy(ei_hbm, ei_v)
        for b in range(E):
            cnt_v[pl.ds(b * L, L)] = zero_v

        def h(i, _):
            idx = iota * CHS + i
            v = plsc.load_gather(ei_v, [idx])
            ci = iota * E + v
            c = plsc.load_gather(cnt_v, [ci])
            plsc.store_scatter(cnt_v, [ci], c + 1)
            return 0
        lax.fori_loop(0, CHS, h, 0)

        def lane_prefix(vec):
            r = vec
            for sh in (1, 2, 4, 8):
                tmp_v[...] = r
                moved = plsc.load_gather(tmp_v, [jnp.maximum(iota - sh, 0)])
                r = r + jnp.where(iota >= sh, moved, 0)
            return r

        g_excl = []
        g_tot = []
        last = zero_v + (L - 1)
        for e in range(E):
            ch = plsc.load_gather(cnt_v, [iota * E + e])
            inc = lane_prefix(ch)
            tmp_v[...] = inc
            g_tot.append(plsc.load_gather(tmp_v, [last]))
            g_excl.append(inc - ch)

        cum = []
        c_val = zero_v
        for e in range(E):
            c_val = c_val + ((g_tot[e] + (T - 1)) >> 7)
            cum.append(c_val)
        starts = [zero_v] + [cum[e] * T for e in range(E - 1)]
        meta = zero_v
        for e in range(E):
            meta = jnp.where(iota == e, g_tot[e], meta)
        meta = jnp.where(iota == E, cum[-1], meta)
        meta_v[...] = meta
        for half in range(2):
            tv = iota + half * L
            acc = zero_v
            for e in range(E):
                acc = acc + jnp.where(tv >= cum[e], 1, 0)
            eid_v[pl.ds(half * L, L)] = jnp.minimum(acc, E - 1)
        for e in range(E):
            plsc.store_scatter(pre_v, [iota * E + e], starts[e] + g_excl[e])
        for b in range(E):
            cnt_v[pl.ds(b * L, L)] = zero_v

        def body(i, _):
            idx = iota * CHS + i
            v = plsc.load_gather(ei_v, [idx])
            ci = iota * E + v
            c = plsc.load_gather(cnt_v, [ci])
            plsc.store_scatter(cnt_v, [ci], c + 1)
            p = plsc.load_gather(pre_v, [ci]) + c
            plsc.store_scatter(dst_v, [idx], p)
            return 0
        lax.fori_loop(0, CHS, body, 0)

        @pl.when(wid == 0)
        def _():
            pltpu.sync_copy(meta_v, meta_hbm)
            pltpu.sync_copy(eid_v, eid_hbm)
            pltpu.sync_copy(dst_v, dst_hbm)

    return k(ei)


def _gather(x, dst):
    TN = N // NW

    @functools.partial(
        pl.kernel,
        out_type=jax.ShapeDtypeStruct((P, D), jnp.float32),
        mesh=_mesh(),
        compiler_params=_SC_PARAMS,
        scratch_types=[pltpu.VMEM((2 * TN,), jnp.int32),
                       pltpu.VMEM((TN,), jnp.int32),
                       pltpu.VMEM((TN,), jnp.int32),
                       pltpu.VMEM((TN, D), jnp.float32),
                       pltpu.SemaphoreType.DMA,
                       pltpu.SemaphoreType.DMA,
                       pltpu.SemaphoreType.DMA],
    )
    def k(x_hbm, dst_hbm, xs_hbm, dst_v, de_v, do_v, rows_v, sr, s0, s1):
        wid = lax.axis_index("s") * 2 + lax.axis_index("c")
        iota = lax.iota(jnp.int32, L)
        rcp = pltpu.async_copy(x_hbm.at[pl.ds(wid * TN, TN)], rows_v, sr)
        pltpu.sync_copy(dst_hbm.at[pl.ds(wid * 2 * TN, 2 * TN)], dst_v)
        for q in range(TN // L):
            de_v[pl.ds(q * L, L)] = plsc.load_gather(
                dst_v, [iota * 2 + q * 2 * L])
            do_v[pl.ds(q * L, L)] = plsc.load_gather(
                dst_v, [iota * 2 + 1 + q * 2 * L])
        rcp.wait()
        c0 = pltpu.async_copy(rows_v, xs_hbm.at[de_v], s0)
        c1 = pltpu.async_copy(rows_v, xs_hbm.at[do_v], s1)
        c0.wait()
        c1.wait()

    return k(x, dst)


def _gemm(eid, meta, xs, w1, w3, w2):
    def body(eid_ref, meta_ref, xs_ref, w1_ref, w3_ref, w2_ref, ys_ref):
        t = pl.program_id(0)

        @pl.when(t < meta_ref[E])
        def _():
            xb = xs_ref[...]
            a = jnp.dot(xb, w1_ref[0], preferred_element_type=jnp.float32)
            b = jnp.dot(xb, w3_ref[0], preferred_element_type=jnp.float32)
            hh = a * jax.nn.sigmoid(a) * b
            y = jnp.dot(hh, w2_ref[0], preferred_element_type=jnp.float32)
            ia = lax.bitcast_convert_type(y[:, :D // 2], jnp.int32)
            ib = lax.bitcast_convert_type(y[:, D // 2:], jnp.int32)

            def rne(iv):
                return lax.shift_right_logical(
                    iv + 0x7FFF
                    + (lax.shift_right_logical(iv, 16) & 1), 16)
            ys_ref[...] = lax.shift_left(rne(ib), 16) | (rne(ia) & 0xFFFF)

    def row_map(t, eid_ref, meta_ref):
        return (jnp.minimum(t, meta_ref[E] - 1), 0)

    def w_map(t, eid_ref, meta_ref):
        return (eid_ref[t], 0, 0)

    grid_spec = pltpu.PrefetchScalarGridSpec(
        num_scalar_prefetch=2,
        grid=(NT_MAX,),
        in_specs=[
            pl.BlockSpec((T, D), row_map),
            pl.BlockSpec((1, D, F), w_map),
            pl.BlockSpec((1, D, F), w_map),
            pl.BlockSpec((1, F, D), w_map),
        ],
        out_specs=pl.BlockSpec((T, D // 2), row_map),
    )
    return pl.pallas_call(
        body,
        grid_spec=grid_spec,
        out_shape=jax.ShapeDtypeStruct((P, D // 2), jnp.int32),
        compiler_params=pltpu.CompilerParams(
            dimension_semantics=("arbitrary",)),
    )(eid, meta, xs, w1, w3, w2)


def _combine(ys, dst, ew):
    TN = N // NW
    CT = 16
    NCH = TN // CT
    DW = D // 2

    @functools.partial(
        pl.kernel,
        out_type=jax.ShapeDtypeStruct((N, D), jnp.float32),
        mesh=_mesh(),
        compiler_params=_SC_PARAMS,
        scratch_types=[pltpu.VMEM((2 * TN,), jnp.int32),
                       pltpu.VMEM((2 * TN,), jnp.float32),
                       pltpu.VMEM((2, 2 * CT, DW), jnp.int32),
                       pltpu.VMEM((2, CT, D), jnp.float32),
                       pltpu.SemaphoreType.DMA,
                       pltpu.SemaphoreType.DMA,
                       pltpu.SemaphoreType.DMA,
                       pltpu.SemaphoreType.DMA],
    )
    def k(ys_hbm, dst_hbm, ew_hbm, out_hbm,
          dst_v, ew_v, r_v, o_v, sg0, sg1, so0, so1):
        wid = lax.axis_index("s") * 2 + lax.axis_index("c")
        iota = lax.iota(jnp.int32, L)
        ab = wid * 2 * TN
        sg = (sg0, sg1)
        so = (so0, so1)
        pltpu.sync_copy(dst_hbm.at[pl.ds(ab, 2 * TN)], dst_v)
        pltpu.sync_copy(ew_hbm.at[pl.ds(ab, 2 * TN)], ew_v)
        zi = jnp.zeros((L,), jnp.int32)
        himask = jnp.full((L,), -65536, jnp.int32)

        def lo(v):
            return plsc.bitcast(lax.shift_left(v, 16), jnp.float32)

        def hi(v):
            return plsc.bitcast(v & himask, jnp.float32)

        gcps = []
        for c in range(2):
            gcps.append(pltpu.async_copy(
                ys_hbm.at[dst_v.at[pl.ds(c * 2 * CT, 2 * CT)]],
                r_v.at[c], sg[c]))
        ocps = [None, None]
        for c in range(NCH):
            bb = c % 2
            gcps[c].wait()
            if ocps[bb] is not None:
                ocps[bb].wait()

            def tok(i, _):
                w0 = plsc.load_gather(ew_v, [zi + (c * 2 * CT + 2 * i)])
                w1v = plsc.load_gather(ew_v, [zi + (c * 2 * CT + 2 * i + 1)])

                def dblk(dd, _):
                    for u in range(4):
                        kk = dd * 4 + u
                        sl = pl.ds(kk * L, L)
                        rv0 = r_v[bb, 2 * i, sl]
                        rv1 = r_v[bb, 2 * i + 1, sl]
                        o_v[bb, i, sl] = w0 * lo(rv0) + w1v * lo(rv1)
                        o_v[bb, i, pl.ds(kk * L + D // 2, L)] = (
                            w0 * hi(rv0) + w1v * hi(rv1))
                    return 0
                lax.fori_loop(0, DW // (4 * L), dblk, 0)
                return 0
            lax.fori_loop(0, CT, tok, 0)
            ocps[bb] = pltpu.async_copy(
                o_v.at[bb], out_hbm.at[pl.ds(wid * TN + c * CT, CT)], so[bb])
            nxt = c + 2
            if nxt < NCH:
                gcps.append(pltpu.async_copy(
                    ys_hbm.at[dst_v.at[pl.ds(nxt * 2 * CT, 2 * CT)]],
                    r_v.at[bb], sg[bb]))
        for bb in range(2):
            if ocps[bb] is not None:
                ocps[bb].wait()

    return k(ys, dst, ew)


@jax.jit
def kernel(x, expert_weights, expert_indices, w1, w2, w3):
    ei = expert_indices.reshape(-1).astype(jnp.int32)
    ew = expert_weights.reshape(-1).astype(jnp.float32)
    meta, eid, dst = _routing(ei)
    xs = _gather(x, dst)
    ys = _gemm(eid, meta, xs, w1, w3, w2)
    out = _combine(ys, dst, ew)
    return out, meta[:E]


# speedup vs baseline: 1.3324x; 1.3324x over previous
import functools

import jax
import jax.numpy as jnp
from jax import lax
from jax.experimental import pallas as pl
from jax.experimental.pallas import tpu as pltpu
from jax.experimental.pallas import tpu_sc as plsc

E = 8
K = 2
D = 1024
F = 2048
N = 2048
A = N * K
T = 256
NT_MAX = A // T + E
P = NT_MAX * T
L = 16
NW = 32


def _mesh():
    return plsc.VectorSubcoreMesh(core_axis_name="c", subcore_axis_name="s")


_SC_PARAMS = pltpu.CompilerParams(needs_layout_passes=False)


def _routing(ei):
    CHS = A // L

    @functools.partial(
        pl.kernel,
        out_type=(jax.ShapeDtypeStruct((L,), jnp.int32),
                  jax.ShapeDtypeStruct((2 * L,), jnp.int32),
                  jax.ShapeDtypeStruct((A,), jnp.int32)),
        mesh=_mesh(),
        compiler_params=_SC_PARAMS,
        scratch_types=[pltpu.VMEM((A,), jnp.int32),
                       pltpu.VMEM((A,), jnp.int32),
                       pltpu.VMEM((L,), jnp.int32),
                       pltpu.VMEM((2 * L,), jnp.int32),
                       pltpu.VMEM((L * E,), jnp.int32),
                       pltpu.VMEM((L * E,), jnp.int32),
                       pltpu.VMEM((L,), jnp.int32),
                       pltpu.SemaphoreType.DMA],
    )
    def k(ei_hbm, meta_hbm, eid_hbm, dst_hbm,
          ei_v, dst_v, meta_v, eid_v, cnt_v, pre_v, tmp_v, sem):
        wid = lax.axis_index("s") * 2 + lax.axis_index("c")
        iota = lax.iota(jnp.int32, L)
        zero_v = jnp.zeros((L,), jnp.int32)

        pltpu.sync_copy(ei_hbm, ei_v)
        for b in range(E):
            cnt_v[pl.ds(b * L, L)] = zero_v

        def h(i, _):
            idx = iota * CHS + i
            v = plsc.load_gather(ei_v, [idx])
            ci = iota * E + v
            c = plsc.load_gather(cnt_v, [ci])
            plsc.store_scatter(cnt_v, [ci], c + 1)
            return 0
        lax.fori_loop(0, CHS, h, 0)

        def lane_prefix(vec):
            r = vec
            for sh in (1, 2, 4, 8):
                tmp_v[...] = r
                moved = plsc.load_gather(tmp_v, [jnp.maximum(iota - sh, 0)])
                r = r + jnp.where(iota >= sh, moved, 0)
            return r

        g_excl = []
        g_tot = []
        last = zero_v + (L - 1)
        for e in range(E):
            ch = plsc.load_gather(cnt_v, [iota * E + e])
            inc = lane_prefix(ch)
            tmp_v[...] = inc
            g_tot.append(plsc.load_gather(tmp_v, [last]))
            g_excl.append(inc - ch)

        cum = []
        c_val = zero_v
        for e in range(E):
            c_val = c_val + ((g_tot[e] + (T - 1)) >> 8)
            cum.append(c_val)
        starts = [zero_v] + [cum[e] * T for e in range(E - 1)]
        meta = zero_v
        for e in range(E):
            meta = jnp.where(iota == e, g_tot[e], meta)
        meta = jnp.where(iota == E, cum[-1], meta)
        meta_v[...] = meta
        for half in range(2):
            tv = iota + half * L
            acc = zero_v
            for e in range(E):
                acc = acc + jnp.where(tv >= cum[e], 1, 0)
            eid_v[pl.ds(half * L, L)] = jnp.minimum(acc, E - 1)
        for e in range(E):
            plsc.store_scatter(pre_v, [iota * E + e], starts[e] + g_excl[e])
        for b in range(E):
            cnt_v[pl.ds(b * L, L)] = zero_v

        def body(i, _):
            idx = iota * CHS + i
            v = plsc.load_gather(ei_v, [idx])
            ci = iota * E + v
            c = plsc.load_gather(cnt_v, [ci])
            plsc.store_scatter(cnt_v, [ci], c + 1)
            p = plsc.load_gather(pre_v, [ci]) + c
            plsc.store_scatter(dst_v, [idx], p)
            return 0
        lax.fori_loop(0, CHS, body, 0)

        @pl.when(wid == 0)
        def _():
            pltpu.sync_copy(meta_v, meta_hbm)
            pltpu.sync_copy(eid_v, eid_hbm)
            pltpu.sync_copy(dst_v, dst_hbm)

    return k(ei)


def _gather(x, dst):
    TN = N // NW

    @functools.partial(
        pl.kernel,
        out_type=jax.ShapeDtypeStruct((P, D), jnp.float32),
        mesh=_mesh(),
        compiler_params=_SC_PARAMS,
        scratch_types=[pltpu.VMEM((2 * TN,), jnp.int32),
                       pltpu.VMEM((TN,), jnp.int32),
                       pltpu.VMEM((TN,), jnp.int32),
                       pltpu.VMEM((TN, D), jnp.float32),
                       pltpu.SemaphoreType.DMA,
                       pltpu.SemaphoreType.DMA,
                       pltpu.SemaphoreType.DMA],
    )
    def k(x_hbm, dst_hbm, xs_hbm, dst_v, de_v, do_v, rows_v, sr, s0, s1):
        wid = lax.axis_index("s") * 2 + lax.axis_index("c")
        iota = lax.iota(jnp.int32, L)
        rcp = pltpu.async_copy(x_hbm.at[pl.ds(wid * TN, TN)], rows_v, sr)
        pltpu.sync_copy(dst_hbm.at[pl.ds(wid * 2 * TN, 2 * TN)], dst_v)
        for q in range(TN // L):
            de_v[pl.ds(q * L, L)] = plsc.load_gather(
                dst_v, [iota * 2 + q * 2 * L])
            do_v[pl.ds(q * L, L)] = plsc.load_gather(
                dst_v, [iota * 2 + 1 + q * 2 * L])
        rcp.wait()
        c0 = pltpu.async_copy(rows_v, xs_hbm.at[de_v], s0)
        c1 = pltpu.async_copy(rows_v, xs_hbm.at[do_v], s1)
        c0.wait()
        c1.wait()

    return k(x, dst)


def _gemm(eid, meta, xs, w1, w3, w2):
    def body(eid_ref, meta_ref, xs_ref, w1_ref, w3_ref, w2_ref, ys_ref):
        t = pl.program_id(0)

        @pl.when(t < meta_ref[E])
        def _():
            xb = xs_ref[...]
            a = jnp.dot(xb, w1_ref[0], preferred_element_type=jnp.float32)
            b = jnp.dot(xb, w3_ref[0], preferred_element_type=jnp.float32)
            hh = a * jax.nn.sigmoid(a) * b
            y = jnp.dot(hh, w2_ref[0], preferred_element_type=jnp.float32)
            ia = lax.bitcast_convert_type(y[:, :D // 2], jnp.int32)
            ib = lax.bitcast_convert_type(y[:, D // 2:], jnp.int32)

            def rne(iv):
                return lax.shift_right_logical(
                    iv + 0x7FFF
                    + (lax.shift_right_logical(iv, 16) & 1), 16)
            ys_ref[...] = lax.shift_left(rne(ib), 16) | (rne(ia) & 0xFFFF)

    def row_map(t, eid_ref, meta_ref):
        return (jnp.minimum(t, meta_ref[E] - 1), 0)

    def w_map(t, eid_ref, meta_ref):
        return (eid_ref[t], 0, 0)

    grid_spec = pltpu.PrefetchScalarGridSpec(
        num_scalar_prefetch=2,
        grid=(NT_MAX,),
        in_specs=[
            pl.BlockSpec((T, D), row_map),
            pl.BlockSpec((1, D, F), w_map),
            pl.BlockSpec((1, D, F), w_map),
            pl.BlockSpec((1, F, D), w_map),
        ],
        out_specs=pl.BlockSpec((T, D // 2), row_map),
    )
    return pl.pallas_call(
        body,
        grid_spec=grid_spec,
        out_shape=jax.ShapeDtypeStruct((P, D // 2), jnp.int32),
        compiler_params=pltpu.CompilerParams(
            dimension_semantics=("arbitrary",)),
    )(eid, meta, xs, w1, w3, w2)


def _combine(ys, dst, ew):
    TN = N // NW
    CT = 16
    NCH = TN // CT
    DW = D // 2

    @functools.partial(
        pl.kernel,
        out_type=jax.ShapeDtypeStruct((N, D), jnp.float32),
        mesh=_mesh(),
        compiler_params=_SC_PARAMS,
        scratch_types=[pltpu.VMEM((2 * TN,), jnp.int32),
                       pltpu.VMEM((2 * TN,), jnp.float32),
                       pltpu.VMEM((2, 2 * CT, DW), jnp.int32),
                       pltpu.VMEM((2, CT, D), jnp.float32),
                       pltpu.SemaphoreType.DMA,
                       pltpu.SemaphoreType.DMA,
                       pltpu.SemaphoreType.DMA,
                       pltpu.SemaphoreType.DMA],
    )
    def k(ys_hbm, dst_hbm, ew_hbm, out_hbm,
          dst_v, ew_v, r_v, o_v, sg0, sg1, so0, so1):
        wid = lax.axis_index("s") * 2 + lax.axis_index("c")
        iota = lax.iota(jnp.int32, L)
        ab = wid * 2 * TN
        sg = (sg0, sg1)
        so = (so0, so1)
        pltpu.sync_copy(dst_hbm.at[pl.ds(ab, 2 * TN)], dst_v)
        pltpu.sync_copy(ew_hbm.at[pl.ds(ab, 2 * TN)], ew_v)
        zi = jnp.zeros((L,), jnp.int32)
        himask = jnp.full((L,), -65536, jnp.int32)

        def lo(v):
            return plsc.bitcast(lax.shift_left(v, 16), jnp.float32)

        def hi(v):
            return plsc.bitcast(v & himask, jnp.float32)

        gcps = []
        for c in range(2):
            gcps.append(pltpu.async_copy(
                ys_hbm.at[dst_v.at[pl.ds(c * 2 * CT, 2 * CT)]],
                r_v.at[c], sg[c]))
        ocps = [None, None]
        for c in range(NCH):
            bb = c % 2
            gcps[c].wait()
            if ocps[bb] is not None:
                ocps[bb].wait()

            def tok(i, _):
                w0 = plsc.load_gather(ew_v, [zi + (c * 2 * CT + 2 * i)])
                w1v = plsc.load_gather(ew_v, [zi + (c * 2 * CT + 2 * i + 1)])

                def dblk(dd, _):
                    for u in range(4):
                        kk = dd * 4 + u
                        sl = pl.ds(kk * L, L)
                        rv0 = r_v[bb, 2 * i, sl]
                        rv1 = r_v[bb, 2 * i + 1, sl]
                        o_v[bb, i, sl] = w0 * lo(rv0) + w1v * lo(rv1)
                        o_v[bb, i, pl.ds(kk * L + D // 2, L)] = (
                            w0 * hi(rv0) + w1v * hi(rv1))
                    return 0
                lax.fori_loop(0, DW // (4 * L), dblk, 0)
                return 0
            lax.fori_loop(0, CT, tok, 0)
            ocps[bb] = pltpu.async_copy(
                o_v.at[bb], out_hbm.at[pl.ds(wid * TN + c * CT, CT)], so[bb])
            nxt = c + 2
            if nxt < NCH:
                gcps.append(pltpu.async_copy(
                    ys_hbm.at[dst_v.at[pl.ds(nxt * 2 * CT, 2 * CT)]],
                    r_v.at[bb], sg[bb]))
        for bb in range(2):
            if ocps[bb] is not None:
                ocps[bb].wait()

    return k(ys, dst, ew)


@jax.jit
def kernel(x, expert_weights, expert_indices, w1, w2, w3):
    ei = expert_indices.reshape(-1).astype(jnp.int32)
    ew = expert_weights.reshape(-1).astype(jnp.float32)
    meta, eid, dst = _routing(ei)
    xs = _gather(x, dst)
    ys = _gemm(eid, meta, xs, w1, w3, w2)
    out = _combine(ys, dst, ew)
    return out, meta[:E]
